# baseline (speedup 1.0000x reference)
import functools

import jax
import jax.numpy as jnp
from jax import lax
from jax.experimental import pallas as pl
from jax.experimental.pallas import tpu as pltpu
from jax.experimental.pallas import tpu_sc as plsc

_N = 10000
_E = 160000
_D = 256
_DH = 128
_ALPHA = 0.5

_NS = 16
_PER_TILE = _E // _NS
_CHP = 128
_NCHP = 80
_PAD_PER_TILE = _NCHP * _CHP - _PER_TILE
_NBUF = 2
_NPAD = 10240
_ROWS_PER_TILE = _NPAD // _NS


def _sc_aggregate(x0, x1, gidx, sidx, zeros):
    mesh = plsc.VectorSubcoreMesh(core_axis_name="c", subcore_axis_name="s")

    @functools.partial(
        pl.kernel,
        out_type=jax.ShapeDtypeStruct((2, 2, _NPAD, _DH), jnp.float32),
        mesh=mesh,
        scratch_types=[
            pltpu.VMEM((_NCHP, _CHP), jnp.int32),
            [pltpu.VMEM((_CHP,), jnp.int32) for _ in range(_NBUF)],
            [pltpu.VMEM((_CHP,), jnp.int32) for _ in range(_NBUF)],
            [pltpu.VMEM((_CHP, _DH), jnp.float32) for _ in range(_NBUF)],
            pltpu.VMEM_SHARED((_NPAD, _DH), jnp.float32),
            [pltpu.SemaphoreType.DMA for _ in range(_NBUF)],
            [pltpu.SemaphoreType.DMA for _ in range(_NBUF)],
        ],
    )
    def agg_kernel(x0_hbm, x1_hbm, gidx_hbm, sidx_hbm, z_hbm, out_hbm,
                   gidx_v, gi, si, rows, acc, gsem, isem):
        c = lax.axis_index("c")
        s = lax.axis_index("s")
        row0 = s * _ROWS_PER_TILE
        sbase = (c * _NS + s) * _NCHP * _CHP

        def copy_row(row, dst1d):
            for k in range(_CHP // 16):
                dst1d[pl.ds(k * 16, 16)] = gidx_v[row, pl.ds(k * 16, 16)]

        def si_copy(i, b):
            return pltpu.make_async_copy(
                sidx_hbm.at[pl.ds(sbase + i * _CHP, _CHP)], si[b], isem[b])

        pltpu.sync_copy(gidx_hbm.at[c, s], gidx_v)
        pltpu.sync_copy(z_hbm, acc.at[pl.ds(row0, _ROWS_PER_TILE)])

        for p, x_hbm in ((0, x0_hbm), (1, x1_hbm)):
            plsc.subcore_barrier()

            for b in range(_NBUF):
                copy_row(b, gi[b])
                pltpu.async_copy(x_hbm.at[gi[b]], rows[b], gsem[b])
                si_copy(b, b).start()

            @pl.loop(0, _NCHP // _NBUF)
            def _(g):
                for b in range(_NBUF):
                    i = g * _NBUF + b
                    pltpu.make_async_copy(
                        x_hbm.at[gi[b]], rows[b], gsem[b]).wait()
                    si_copy(i, b).wait()
                    pltpu.sync_copy(rows[b], acc.at[si[b]], add=True)
                    nxt = i + _NBUF

                    @pl.when(nxt < _NCHP)
                    def _():
                        copy_row(nxt, gi[b])
                        pltpu.async_copy(x_hbm.at[gi[b]], rows[b], gsem[b])
                        si_copy(nxt, b).start()

            plsc.subcore_barrier()
            pltpu.sync_copy(acc.at[pl.ds(row0, _ROWS_PER_TILE)],
                            out_hbm.at[c, p, pl.ds(row0, _ROWS_PER_TILE)])
            if p == 0:
                pltpu.sync_copy(z_hbm, acc.at[pl.ds(row0, _ROWS_PER_TILE)])

    return agg_kernel(x0, x1, gidx, sidx, zeros)


def _arrange_idx(v, pad_val):
    a = v.reshape(_NS, _PER_TILE)
    pad = jnp.full((_NS, _PAD_PER_TILE), pad_val, jnp.int32)
    return jnp.concatenate([a, pad], axis=1).reshape(_NS, _NCHP, _CHP)


_BLK = 1000


def _mlp_body(x_ref, as0_ref, as1_ref, ad0_ref, ad1_ref,
              w1s, b1s, w2s, b2s, w1d, b1d, w2d, b2d, o_ref):
    xs = x_ref[...]
    hs = xs + jnp.concatenate([as0_ref[...], as1_ref[...]], axis=-1)
    hd = xs + jnp.concatenate([ad0_ref[...], ad1_ref[...]], axis=-1)
    ts = jnp.maximum(
        jnp.dot(hs, w1s[...], preferred_element_type=jnp.float32) + b1s[...], 0.0)
    ys = jnp.dot(ts, w2s[...], preferred_element_type=jnp.float32) + b2s[...]
    td = jnp.maximum(
        jnp.dot(hd, w1d[...], preferred_element_type=jnp.float32) + b1d[...], 0.0)
    yd = jnp.dot(td, w2d[...], preferred_element_type=jnp.float32) + b2d[...]
    o_ref[...] = (1.0 - _ALPHA) * ys + _ALPHA * yd


def _tc_mlp(x, aggs0, aggs1, aggd0, aggd1,
            W1s, b1s, W2s, b2s, W1d, b1d, W2d, b2d):
    half_spec = pl.BlockSpec((_BLK, _DH), lambda i: (i, 0))
    w_spec = pl.BlockSpec((_D, _D), lambda i: (0, 0))
    b_spec = pl.BlockSpec((1, _D), lambda i: (0, 0))
    return pl.pallas_call(
        _mlp_body,
        grid=(_N // _BLK,),
        in_specs=[
            pl.BlockSpec((_BLK, _D), lambda i: (i, 0)),
            half_spec, half_spec, half_spec, half_spec,
            w_spec, b_spec, w_spec, b_spec,
            w_spec, b_spec, w_spec, b_spec,
        ],
        out_specs=pl.BlockSpec((_BLK, _D), lambda i: (i, 0)),
        out_shape=jax.ShapeDtypeStruct((_N, _D), jnp.float32),
    )(x, aggs0, aggs1, aggd0, aggd1,
      W1s, b1s.reshape(1, _D), W2s, b2s.reshape(1, _D),
      W1d, b1d.reshape(1, _D), W2d, b2d.reshape(1, _D))


def kernel(x, edge_index, W1s, b1s, W2s, b2s, W1d, b1d, W2d, b2d):
    x0 = x[:, :_DH]
    x1 = x[:, _DH:]
    src = edge_index[0]
    dst = edge_index[1]
    gidx = jnp.stack([_arrange_idx(src, 0), _arrange_idx(dst, 0)])
    sidx = jnp.stack([_arrange_idx(dst, _N), _arrange_idx(src, _N)])
    zeros = jnp.zeros((_ROWS_PER_TILE, _DH), jnp.float32)
    agg = _sc_aggregate(x0, x1, gidx, sidx.reshape(-1), zeros)
    return _tc_mlp(x, agg[0, 0, :_N], agg[0, 1, :_N], agg[1, 0, :_N],
                   agg[1, 1, :_N],
                   W1s, b1s, W2s, b2s, W1d, b1d, W2d, b2d)
